# baseline (speedup 1.0000x reference)
import functools

import jax
import jax.numpy as jnp
from jax import lax
from jax.experimental import pallas as pl
from jax.experimental.pallas import tpu as pltpu
from jax.experimental.pallas import tpu_sc as plsc

B = 16
PS = 4096
C = 16
N = 65536

IN_WORDS = B * PS * 2 * C
OUT_WORDS = B * N * C
CHUNK_IN = 2048
CHUNK_OUT = 32768
NCHUNK = 16


@functools.partial(
    pl.kernel,
    mesh=plsc.VectorSubcoreMesh(core_axis_name="c", subcore_axis_name="s"),
    out_type=jax.ShapeDtypeStruct((OUT_WORDS,), jnp.float32),
    scratch_types=[
        pltpu.VMEM((CHUNK_IN,), jnp.float32),
        pltpu.VMEM((CHUNK_IN,), jnp.float32),
        pltpu.VMEM((CHUNK_IN,), jnp.float32),
        pltpu.VMEM((CHUNK_IN,), jnp.float32),
        pltpu.VMEM((CHUNK_IN,), jnp.int32),
        pltpu.VMEM((CHUNK_IN,), jnp.int32),
        pltpu.VMEM((CHUNK_OUT,), jnp.float32),
        pltpu.VMEM((CHUNK_OUT,), jnp.float32),
        pltpu.SemaphoreType.DMA,
        pltpu.SemaphoreType.DMA,
        pltpu.SemaphoreType.DMA,
        pltpu.SemaphoreType.DMA,
    ],
    compiler_params=pltpu.CompilerParams(
        needs_layout_passes=False,
        skip_device_barrier=True,
        disable_bounds_checks=True,
        disable_semaphore_checks=True,
    ),
)
def _velvet_sc(in_hbm, out_hbm, sgn_a, frac_a, sgn_b, frac_b, offs_a, offs_b,
               out_a, out_b, sem_a, sem_b, isem_a, isem_b):
    b = lax.axis_index("s")
    cr = lax.axis_index("c")
    w = b * 2 + cr
    sgn_w0 = (b * 4 + cr) * 256 * 128
    frac_w0 = sgn_w0 + 512 * 128
    out_w0 = w * 4096 * 128

    iota = lax.iota(jnp.int32, 16)
    lane_off = lax.shift_left(iota & 8, 7) + lax.shift_left(iota & 7, 4)
    zvec = jnp.zeros((16,), jnp.float32)

    def zero_buf(buf):
        def zrow(r, _):
            base = r * 128
            for cc in range(8):
                buf[pl.ds(base + cc * 16, 16)] = zvec
            return 0
        lax.fori_loop(0, 256, zrow, 0)

    def in_issue(k, sgn_v, frac_v, sem):
        pltpu.async_copy(in_hbm.at[pl.ds(sgn_w0 + CHUNK_IN * k, CHUNK_IN)], sgn_v, sem)
        pltpu.async_copy(in_hbm.at[pl.ds(frac_w0 + CHUNK_IN * k, CHUNK_IN)], frac_v, sem)

    def in_wait(k, sgn_v, frac_v, sem):
        pltpu.make_async_copy(in_hbm.at[pl.ds(sgn_w0 + CHUNK_IN * k, CHUNK_IN)], sgn_v, sem).wait()
        pltpu.make_async_copy(in_hbm.at[pl.ds(frac_w0 + CHUNK_IN * k, CHUNK_IN)], frac_v, sem).wait()

    def value_pass(sgn_v, frac_v, out_v, offs_v):
        def mbody(m, _):
            pbl = m & 1
            c8 = lax.shift_right_logical(m, 1)
            sbase = pbl * 1024 + c8 * 128
            obase0 = pbl * 16384 + c8 * 128
            for gg in range(8):
                sgn = sgn_v[pl.ds(sbase + gg * 16, 16)]
                frac = frac_v[pl.ds(sbase + gg * 16, 16)]
                idx = (frac * 15.0).astype(jnp.int32)
                offs = (lane_off + (obase0 + gg * 2048)) + idx
                offs_v[pl.ds(m * 128 + gg * 16, 16)] = offs
                plsc.store_scatter(out_v, [offs], sgn)
            return 0
        lax.fori_loop(0, 16, mbody, 0)

    def rescatter_zeros(offs_v, out_v):
        def rbody(m, _):
            for gg in range(8):
                offs = offs_v[pl.ds(m * 128 + gg * 16, 16)]
                plsc.store_scatter(out_v, [offs], zvec)
            return 0
        lax.fori_loop(0, 16, rbody, 0)

    def out_slice(k):
        return out_hbm.at[pl.ds(out_w0 + CHUNK_OUT * k, CHUNK_OUT)]

    in_issue(0, sgn_a, frac_a, isem_a)
    in_issue(1, sgn_b, frac_b, isem_b)
    zero_buf(out_a)
    zero_buf(out_b)
    in_wait(0, sgn_a, frac_a, isem_a)
    value_pass(sgn_a, frac_a, out_a, offs_a)
    in_issue(2, sgn_a, frac_a, isem_a)
    pltpu.async_copy(out_a, out_slice(0), sem_a)
    in_wait(1, sgn_b, frac_b, isem_b)
    value_pass(sgn_b, frac_b, out_b, offs_b)
    in_issue(3, sgn_b, frac_b, isem_b)
    pltpu.async_copy(out_b, out_slice(1), sem_b)

    def pair(t, _):
        k0 = 2 * t
        pltpu.make_async_copy(out_a, out_slice(k0 - 2), sem_a).wait()
        rescatter_zeros(offs_a, out_a)
        in_wait(k0, sgn_a, frac_a, isem_a)
        value_pass(sgn_a, frac_a, out_a, offs_a)
        in_issue(k0 + 2, sgn_a, frac_a, isem_a)
        pltpu.async_copy(out_a, out_slice(k0), sem_a)
        pltpu.make_async_copy(out_b, out_slice(k0 - 1), sem_b).wait()
        rescatter_zeros(offs_b, out_b)
        in_wait(k0 + 1, sgn_b, frac_b, isem_b)
        value_pass(sgn_b, frac_b, out_b, offs_b)
        in_issue(k0 + 3, sgn_b, frac_b, isem_b)
        pltpu.async_copy(out_b, out_slice(k0 + 1), sem_b)
        return 0

    lax.fori_loop(1, NCHUNK // 2 - 1, pair, 0)

    pltpu.make_async_copy(out_a, out_slice(NCHUNK - 4), sem_a).wait()
    rescatter_zeros(offs_a, out_a)
    in_wait(NCHUNK - 2, sgn_a, frac_a, isem_a)
    value_pass(sgn_a, frac_a, out_a, offs_a)
    pltpu.async_copy(out_a, out_slice(NCHUNK - 2), sem_a)
    pltpu.make_async_copy(out_b, out_slice(NCHUNK - 3), sem_b).wait()
    rescatter_zeros(offs_b, out_b)
    in_wait(NCHUNK - 1, sgn_b, frac_b, isem_b)
    value_pass(sgn_b, frac_b, out_b, offs_b)
    pltpu.async_copy(out_b, out_slice(NCHUNK - 1), sem_b)

    pltpu.make_async_copy(out_a, out_slice(NCHUNK - 2), sem_a).wait()
    pltpu.make_async_copy(out_b, out_slice(NCHUNK - 1), sem_b).wait()


def kernel(inputs):
    flat_in = (inputs.transpose(0, 2, 1)
               .reshape(B, 4, 8, 32, 128)
               .transpose(0, 1, 3, 2, 4)
               .reshape(IN_WORDS))
    flat_out = _velvet_sc(flat_in)
    return (flat_out.reshape(B, 2, 512, 8, 128)
            .transpose(0, 2, 4, 1, 3)
            .reshape(B, N, C))


# speedup vs baseline: 1.0007x; 1.0007x over previous
import functools

import jax
import jax.numpy as jnp
from jax import lax
from jax.experimental import pallas as pl
from jax.experimental.pallas import tpu as pltpu
from jax.experimental.pallas import tpu_sc as plsc

B = 16
PS = 4096
C = 16
N = 65536

IN_WORDS = B * PS * 2 * C
OUT_WORDS = B * N * C
CHUNK_IN = 2048
CHUNK_OUT = 32768
NCHUNK = 16


@functools.partial(
    pl.kernel,
    mesh=plsc.VectorSubcoreMesh(core_axis_name="c", subcore_axis_name="s"),
    out_type=jax.ShapeDtypeStruct((OUT_WORDS,), jnp.float32),
    scratch_types=[
        pltpu.VMEM((CHUNK_IN,), jnp.float32),
        pltpu.VMEM((CHUNK_IN,), jnp.float32),
        pltpu.VMEM((CHUNK_IN,), jnp.float32),
        pltpu.VMEM((CHUNK_IN,), jnp.float32),
        pltpu.VMEM((CHUNK_IN,), jnp.int32),
        pltpu.VMEM((CHUNK_IN,), jnp.int32),
        pltpu.VMEM((CHUNK_OUT,), jnp.float32),
        pltpu.VMEM((CHUNK_OUT,), jnp.float32),
        pltpu.SemaphoreType.DMA,
        pltpu.SemaphoreType.DMA,
        pltpu.SemaphoreType.DMA,
        pltpu.SemaphoreType.DMA,
    ],
    compiler_params=pltpu.CompilerParams(needs_layout_passes=False),
)
def _velvet_sc(in_hbm, out_hbm, sgn_a, frac_a, sgn_b, frac_b, offs_a, offs_b,
               out_a, out_b, sem_a, sem_b, isem_a, isem_b):
    b = lax.axis_index("s")
    cr = lax.axis_index("c")
    w = b * 2 + cr
    sgn_w0 = (b * 4 + cr) * 256 * 128
    frac_w0 = sgn_w0 + 512 * 128
    out_w0 = w * 4096 * 128

    iota = lax.iota(jnp.int32, 16)
    lane_off = lax.shift_left(iota & 8, 7) + lax.shift_left(iota & 7, 4)
    zvec = jnp.zeros((16,), jnp.float32)

    def zero_buf(buf):
        def zrow(r, _):
            base = r * 128
            for cc in range(8):
                buf[pl.ds(base + cc * 16, 16)] = zvec
            return 0
        lax.fori_loop(0, 256, zrow, 0)

    def in_issue(k, sgn_v, frac_v, sem):
        pltpu.async_copy(in_hbm.at[pl.ds(sgn_w0 + CHUNK_IN * k, CHUNK_IN)], sgn_v, sem)
        pltpu.async_copy(in_hbm.at[pl.ds(frac_w0 + CHUNK_IN * k, CHUNK_IN)], frac_v, sem)

    def in_wait(k, sgn_v, frac_v, sem):
        pltpu.make_async_copy(in_hbm.at[pl.ds(sgn_w0 + CHUNK_IN * k, CHUNK_IN)], sgn_v, sem).wait()
        pltpu.make_async_copy(in_hbm.at[pl.ds(frac_w0 + CHUNK_IN * k, CHUNK_IN)], frac_v, sem).wait()

    def value_pass(sgn_v, frac_v, out_v, offs_v):
        def mbody(m, _):
            pbl = m & 1
            c8 = lax.shift_right_logical(m, 1)
            sbase = pbl * 1024 + c8 * 128
            obase0 = pbl * 16384 + c8 * 128
            for gg in range(8):
                sgn = sgn_v[pl.ds(sbase + gg * 16, 16)]
                frac = frac_v[pl.ds(sbase + gg * 16, 16)]
                idx = (frac * 15.0).astype(jnp.int32)
                offs = (lane_off + (obase0 + gg * 2048)) + idx
                offs_v[pl.ds(m * 128 + gg * 16, 16)] = offs
                plsc.store_scatter(out_v, [offs], sgn)
            return 0
        lax.fori_loop(0, 16, mbody, 0)

    def rescatter_zeros(offs_v, out_v):
        def rbody(m, _):
            for gg in range(8):
                offs = offs_v[pl.ds(m * 128 + gg * 16, 16)]
                plsc.store_scatter(out_v, [offs], zvec)
            return 0
        lax.fori_loop(0, 16, rbody, 0)

    def out_slice(k):
        return out_hbm.at[pl.ds(out_w0 + CHUNK_OUT * k, CHUNK_OUT)]

    in_issue(0, sgn_a, frac_a, isem_a)
    in_issue(1, sgn_b, frac_b, isem_b)
    zero_buf(out_a)
    in_wait(0, sgn_a, frac_a, isem_a)
    value_pass(sgn_a, frac_a, out_a, offs_a)
    in_issue(2, sgn_a, frac_a, isem_a)
    pltpu.async_copy(out_a, out_slice(0), sem_a)
    zero_buf(out_b)
    in_wait(1, sgn_b, frac_b, isem_b)
    value_pass(sgn_b, frac_b, out_b, offs_b)
    in_issue(3, sgn_b, frac_b, isem_b)
    pltpu.async_copy(out_b, out_slice(1), sem_b)

    def pair(t, _):
        k0 = 2 * t
        pltpu.make_async_copy(out_a, out_slice(k0 - 2), sem_a).wait()
        rescatter_zeros(offs_a, out_a)
        in_wait(k0, sgn_a, frac_a, isem_a)
        value_pass(sgn_a, frac_a, out_a, offs_a)
        in_issue(k0 + 2, sgn_a, frac_a, isem_a)
        pltpu.async_copy(out_a, out_slice(k0), sem_a)
        pltpu.make_async_copy(out_b, out_slice(k0 - 1), sem_b).wait()
        rescatter_zeros(offs_b, out_b)
        in_wait(k0 + 1, sgn_b, frac_b, isem_b)
        value_pass(sgn_b, frac_b, out_b, offs_b)
        in_issue(k0 + 3, sgn_b, frac_b, isem_b)
        pltpu.async_copy(out_b, out_slice(k0 + 1), sem_b)
        return 0

    lax.fori_loop(1, NCHUNK // 2 - 1, pair, 0)

    pltpu.make_async_copy(out_a, out_slice(NCHUNK - 4), sem_a).wait()
    rescatter_zeros(offs_a, out_a)
    in_wait(NCHUNK - 2, sgn_a, frac_a, isem_a)
    value_pass(sgn_a, frac_a, out_a, offs_a)
    pltpu.async_copy(out_a, out_slice(NCHUNK - 2), sem_a)
    pltpu.make_async_copy(out_b, out_slice(NCHUNK - 3), sem_b).wait()
    rescatter_zeros(offs_b, out_b)
    in_wait(NCHUNK - 1, sgn_b, frac_b, isem_b)
    value_pass(sgn_b, frac_b, out_b, offs_b)
    pltpu.async_copy(out_b, out_slice(NCHUNK - 1), sem_b)

    pltpu.make_async_copy(out_a, out_slice(NCHUNK - 2), sem_a).wait()
    pltpu.make_async_copy(out_b, out_slice(NCHUNK - 1), sem_b).wait()


def kernel(inputs):
    flat_in = (inputs.transpose(0, 2, 1)
               .reshape(B, 4, 8, 32, 128)
               .transpose(0, 1, 3, 2, 4)
               .reshape(IN_WORDS))
    flat_out = _velvet_sc(flat_in)
    return (flat_out.reshape(B, 2, 512, 8, 128)
            .transpose(0, 2, 4, 1, 3)
            .reshape(B, N, C))
